# Initial kernel scaffold; baseline (speedup 1.0000x reference)
import jax
import jax.numpy as jnp
from jax.experimental import pallas as pl


def kernel(input, indices):
    raise NotImplementedError("write your pallas kernel here")


# speedup vs baseline: 4.0501x; 4.0501x over previous
import jax
import jax.numpy as jnp
from jax import lax
from jax.experimental import pallas as pl
from jax.experimental.pallas import tpu as pltpu
from jax.experimental.pallas import tpu_sc as plsc

_B, _C, _T, _H, _W = 4, 64, 16, 32, 32
_OT, _OH, _OW = 32, 64, 64
_NS = _B * _C
_E = _T * _H * _W
_OWF = _OT * _OH * _OW
_HALF = _OWF // 2
_TOT = _NS * _E
_NC, _NSUB = 2, 16
_NW = _NC * _NSUB
_RHPW = 2 * _NS // _NW
_PAD = _E
_NBND = 2 * _NS + 1


def _body(sgi_hbm, sval_hbm, bnd_hbm, out_hbm, seg_i, seg_v, half_v, bnd_v):
    wid = lax.axis_index("s") * _NC + lax.axis_index("c")
    pltpu.sync_copy(bnd_hbm, bnd_v)
    lanes = lax.iota(jnp.int32, 16)

    def bnd_at(j):
        chunk = bnd_v[pl.ds((j // 16) * 16, 16)]
        return jnp.max(jnp.where(lanes == j % 16, chunk,
                                 jnp.int32(-2147483648)))

    def one_rh(k, carry):
        rh = wid * _RHPW + k
        row = rh // 2
        lo = (rh % 2) * _HALF
        base = rh * _HALF
        start = bnd_at(rh)
        start_a = (start // 16) * 16
        nv = (bnd_at(rh + 1) - start_a + 15) // 16

        def zero_v(i, c):
            half_v[pl.ds(i * 16, 16)] = jnp.zeros((16,), jnp.float32)
            return c

        lax.fori_loop(0, _HALF // 16, zero_v, None)
        pltpu.sync_copy(sgi_hbm.at[pl.ds(start_a, _E)], seg_i)
        pltpu.sync_copy(sval_hbm.at[pl.ds(start_a, _E)], seg_v)

        def scat_v(v, c):
            gi = seg_i[pl.ds(v * 16, 16)]
            val = seg_v[pl.ds(v * 16, 16)]
            li = gi - base
            m = (li >= 0) & (li < _HALF)
            _, last = plsc.scan_count(gi)
            m = m & last
            si = jnp.where(m, li, 0)
            plsc.store_scatter(half_v, [si], val, mask=m)
            return c

        lax.fori_loop(0, nv, scat_v, None)
        pltpu.sync_copy(half_v, out_hbm.at[row, pl.ds(lo, _HALF)])
        return carry

    lax.fori_loop(0, _RHPW, one_rh, None)


_sc_call = pl.kernel(
    _body,
    out_type=jax.ShapeDtypeStruct((_NS, _OWF), jnp.float32),
    mesh=plsc.VectorSubcoreMesh(core_axis_name="c", subcore_axis_name="s"),
    compiler_params=pltpu.CompilerParams(needs_layout_passes=False),
    scratch_types=[
        pltpu.VMEM((_E,), jnp.int32),
        pltpu.VMEM((_E,), jnp.float32),
        pltpu.VMEM((_HALF,), jnp.float32),
        pltpu.VMEM((_NBND + 15,), jnp.int32),
    ],
)


def kernel(input, indices):
    x = input.reshape(_TOT)
    idx2 = indices.reshape(_NS, _E)
    rows = jnp.arange(_NS, dtype=jnp.int32)[:, None]
    gidx = (rows * _OWF + idx2).reshape(_TOT)
    sgi, sv = lax.sort((gidx, x), dimension=0, is_stable=False, num_keys=1)
    bnd = jnp.searchsorted(
        sgi, jnp.arange(_NBND, dtype=jnp.int32) * _HALF).astype(jnp.int32)
    bnd = jnp.concatenate([bnd, jnp.full((15,), _TOT, jnp.int32)])
    sgi_p = jnp.concatenate(
        [sgi, jnp.full((_PAD,), jnp.int32(0x7FFFFFFF))])
    sv_p = jnp.concatenate([sv, jnp.zeros((_PAD,), jnp.float32)])
    out = _sc_call(sgi_p, sv_p, bnd)
    return out.reshape(_B, _C, _OT, _OH, _OW)


# speedup vs baseline: 4.3029x; 1.0624x over previous
import jax
import jax.numpy as jnp
from jax import lax
from jax.experimental import pallas as pl
from jax.experimental.pallas import tpu as pltpu
from jax.experimental.pallas import tpu_sc as plsc

_B, _C, _T, _H, _W = 4, 64, 16, 32, 32
_OT, _OH, _OW = 32, 64, 64
_NS = _B * _C
_E = _T * _H * _W
_OWF = _OT * _OH * _OW
_HALF = _OWF // 2
_TOT = _NS * _E
_NC, _NSUB = 2, 16
_NW = _NC * _NSUB
_RHPW = 2 * _NS // _NW
_SEG = 16640
_NBND = 2 * _NS + 1


def _body(sgi_hbm, sval_hbm, bnd_hbm, out_hbm, seg_i, seg_v, half_v, bnd_v):
    wid = lax.axis_index("s") * _NC + lax.axis_index("c")
    pltpu.sync_copy(bnd_hbm, bnd_v)
    lanes = lax.iota(jnp.int32, 16)
    zeros16 = jnp.zeros((16,), jnp.float32)

    def bnd_at(j):
        chunk = bnd_v[pl.ds((j // 16) * 16, 16)]
        return jnp.max(jnp.where(lanes == j % 16, chunk,
                                 jnp.int32(-2147483648)))

    def one_rh(k, carry):
        rh = wid * _RHPW + k
        row = rh // 2
        lo = (rh % 2) * _HALF
        base = rh * _HALF
        start_a = jnp.minimum((bnd_at(rh) // 16) * 16, _TOT - _SEG)
        nv8 = (bnd_at(rh + 1) - start_a + 127) // 128

        def zero_v(i, c):
            for j in range(8):
                half_v[pl.ds((i * 8 + j) * 16, 16)] = zeros16
            return c

        lax.fori_loop(0, _HALF // 128, zero_v, None)
        pltpu.sync_copy(sgi_hbm.at[pl.ds(start_a, _SEG)], seg_i)
        pltpu.sync_copy(sval_hbm.at[pl.ds(start_a, _SEG)], seg_v)

        def scat_v(v, c):
            for j in range(8):
                off = (v * 8 + j) * 16
                gi = seg_i[pl.ds(off, 16)]
                val = seg_v[pl.ds(off, 16)]
                li = gi - base
                m = (li >= 0) & (li < _HALF)
                _, last = plsc.scan_count(gi)
                m = m & last
                si = jnp.where(m, li, 0)
                plsc.store_scatter(half_v, [si], val, mask=m)
            return c

        lax.fori_loop(0, nv8, scat_v, None)
        pltpu.sync_copy(half_v, out_hbm.at[row, pl.ds(lo, _HALF)])
        return carry

    lax.fori_loop(0, _RHPW, one_rh, None)


_sc_call = pl.kernel(
    _body,
    out_type=jax.ShapeDtypeStruct((_NS, _OWF), jnp.float32),
    mesh=plsc.VectorSubcoreMesh(core_axis_name="c", subcore_axis_name="s"),
    compiler_params=pltpu.CompilerParams(needs_layout_passes=False),
    scratch_types=[
        pltpu.VMEM((_SEG,), jnp.int32),
        pltpu.VMEM((_SEG,), jnp.float32),
        pltpu.VMEM((_HALF,), jnp.float32),
        pltpu.VMEM((_NBND + 15,), jnp.int32),
    ],
)


def kernel(input, indices):
    x = input.reshape(_TOT)
    idx2 = indices.reshape(_NS, _E)
    rows = jnp.arange(_NS, dtype=jnp.int32)[:, None]
    gidx = (rows * _OWF + idx2).reshape(_TOT)
    sgi, sv = lax.sort((gidx, x), dimension=0, is_stable=False, num_keys=1)
    queries = jnp.minimum(jnp.arange(_NBND + 15, dtype=jnp.int32),
                          _NBND - 1) * _HALF
    bnd = jnp.searchsorted(sgi, queries).astype(jnp.int32)
    out = _sc_call(sgi, sv, bnd)
    return out.reshape(_B, _C, _OT, _OH, _OW)


# speedup vs baseline: 4.6331x; 1.0767x over previous
import jax
import jax.numpy as jnp
from jax import lax
from jax.experimental import pallas as pl
from jax.experimental.pallas import tpu as pltpu
from jax.experimental.pallas import tpu_sc as plsc

_B, _C, _T, _H, _W = 4, 64, 16, 32, 32
_OT, _OH, _OW = 32, 64, 64
_NS = _B * _C
_E = _T * _H * _W
_OWF = _OT * _OH * _OW
_HALF = _OWF // 2
_TOT = _NS * _E
_NC, _NSUB = 2, 16
_NW = _NC * _NSUB
_RHPW = 2 * _NS // _NW
_SEG = 16640


def _body(sgi_hbm, sval_hbm, out_hbm, seg_i, seg_v, half_v, probe_v, sem):
    wid = lax.axis_index("s") * _NC + lax.axis_index("c")
    lanes = lax.iota(jnp.int32, 16)
    zeros16 = jnp.zeros((16,), jnp.float32)

    def bsearch(q):
        def step(t, lh):
            blo, bhi = lh
            mid = (blo + bhi) // 2
            pltpu.async_copy(sgi_hbm.at[mid], probe_v, sem).wait()
            lt = probe_v[...] < q
            return (jnp.where(lt, mid + 1, blo), jnp.where(lt, bhi, mid))

        blo, bhi = lax.fori_loop(
            0, 22, step,
            (jnp.zeros((16,), jnp.int32), jnp.full((16,), _TOT, jnp.int32)))
        return blo

    b0 = bsearch((wid * _RHPW + lanes) * _HALF)
    b1 = bsearch((wid * _RHPW + lanes + 1) * _HALF)

    def one_rh(k, carry):
        b0c, b1c = carry
        rh = wid * _RHPW + k
        row = rh // 2
        lo = (rh % 2) * _HALF
        base = rh * _HALF
        neg = jnp.int32(-2147483648)
        start = jnp.max(jnp.where(lanes == k, b0c, neg))
        end = jnp.max(jnp.where(lanes == k, b1c, neg))
        start_a = jnp.minimum((start // 16) * 16, _TOT - _SEG)
        nv8 = (end - start_a + 127) // 128

        ci = pltpu.async_copy(sgi_hbm.at[pl.ds(start_a, _SEG)], seg_i, sem)
        cv = pltpu.async_copy(sval_hbm.at[pl.ds(start_a, _SEG)], seg_v, sem)

        def zero_v(i, c):
            for j in range(8):
                half_v[pl.ds((i * 8 + j) * 16, 16)] = zeros16
            return c

        lax.fori_loop(0, _HALF // 128, zero_v, None)
        ci.wait()
        cv.wait()

        def scat_v(v, c):
            for j in range(8):
                off = (v * 8 + j) * 16
                gi = seg_i[pl.ds(off, 16)]
                val = seg_v[pl.ds(off, 16)]
                li = gi - base
                m = (li >= 0) & (li < _HALF)
                _, last = plsc.scan_count(gi)
                m = m & last
                si = jnp.where(m, li, 0)
                plsc.store_scatter(half_v, [si], val, mask=m)
            return c

        lax.fori_loop(0, nv8, scat_v, None)
        pltpu.sync_copy(half_v, out_hbm.at[row, pl.ds(lo, _HALF)])
        return (b0c, b1c)

    lax.fori_loop(0, _RHPW, one_rh, (b0, b1))


_sc_call = pl.kernel(
    _body,
    out_type=jax.ShapeDtypeStruct((_NS, _OWF), jnp.float32),
    mesh=plsc.VectorSubcoreMesh(core_axis_name="c", subcore_axis_name="s"),
    compiler_params=pltpu.CompilerParams(needs_layout_passes=False),
    scratch_types=[
        pltpu.VMEM((_SEG,), jnp.int32),
        pltpu.VMEM((_SEG,), jnp.float32),
        pltpu.VMEM((_HALF,), jnp.float32),
        pltpu.VMEM((16,), jnp.int32),
        pltpu.SemaphoreType.DMA,
    ],
)


def kernel(input, indices):
    x = input.reshape(_TOT)
    idx2 = indices.reshape(_NS, _E)
    rows = jnp.arange(_NS, dtype=jnp.int32)[:, None]
    gidx = (rows * _OWF + idx2).reshape(_TOT)
    sgi, sv = lax.sort((gidx, x), dimension=0, is_stable=False, num_keys=1)
    out = _sc_call(sgi, sv)
    return out.reshape(_B, _C, _OT, _OH, _OW)
